# baseline (speedup 1.0000x reference)
import functools

import jax
import jax.numpy as jnp
from jax import lax
from jax.experimental import pallas as pl
from jax.experimental.pallas import tpu as pltpu
from jax.experimental.pallas import tpu_sc as plsc

D = 768
H = 12
DH = D // H
E = 8
DFF = 3072
T = 2048
EPS = 1e-5
BLK = 128
NP = T + E * BLK
NB = NP // BLK
NBPAD = 32
TB = 256
RB = 512
SCALE = 1.0 / (DH ** 0.5)


def _proj_body(x_ref, wk, bk, wq, bq, wv, bv, wqi, bqi, wki, bki, wvi, bvi,
               q_ref, k_ref, v_ref):
    x = x_ref[...]
    q_ref[...] = (x @ wk[...] + bk[...]) @ wqi[...] + bqi[...]
    k_ref[...] = (x @ wq[...] + bq[...]) @ wki[...] + bki[...]
    v_ref[...] = (x @ wv[...] + bv[...]) @ wvi[...] + bvi[...]


def _proj(x, p):
    w_spec = pl.BlockSpec((D, D), lambda i: (0, 0))
    b_spec = pl.BlockSpec((1, D), lambda i: (0, 0))
    ts = pl.BlockSpec((TB, D), lambda i: (i, 0))
    out = jax.ShapeDtypeStruct((T, D), jnp.float32)
    return pl.pallas_call(
        _proj_body,
        grid=(T // TB,),
        in_specs=[ts, w_spec, b_spec, w_spec, b_spec, w_spec, b_spec,
                  w_spec, b_spec, w_spec, b_spec, w_spec, b_spec],
        out_specs=[ts, ts, ts],
        out_shape=[out, out, out],
    )(x, p['Wk'], p['bk'].reshape(1, D), p['Wq'], p['bq'].reshape(1, D),
      p['Wv'], p['bv'].reshape(1, D), p['Wqi'], p['bqi'].reshape(1, D),
      p['Wki'], p['bki'].reshape(1, D), p['Wvi'], p['bvi'].reshape(1, D))


def _attn_body(q_ref, k_ref, v_ref, o_ref):
    q = q_ref[0] * SCALE
    s = lax.dot_general(q, k_ref[0], (((1,), (1,)), ((), ())),
                        preferred_element_type=jnp.float32)
    m = jnp.max(s, axis=-1, keepdims=True)
    e = jnp.exp(s - m)
    z = jnp.sum(e, axis=-1, keepdims=True)
    o = lax.dot_general(e, v_ref[0], (((1,), (0,)), ((), ())),
                        preferred_element_type=jnp.float32)
    o_ref[0] = o / z


def _attn(q, k, v):
    return pl.pallas_call(
        _attn_body,
        grid=(H, T // RB),
        in_specs=[pl.BlockSpec((1, RB, DH), lambda h, i: (h, i, 0)),
                  pl.BlockSpec((1, T, DH), lambda h, i: (h, 0, 0)),
                  pl.BlockSpec((1, T, DH), lambda h, i: (h, 0, 0))],
        out_specs=pl.BlockSpec((1, RB, DH), lambda h, i: (h, i, 0)),
        out_shape=jax.ShapeDtypeStruct((H, T, DH), jnp.float32),
    )(q, k, v)


def _post_body(x_ref, o_ref, wo, bo, g, b, ws, bs,
               h_ref, routes_ref, pmax_ref, stats_ref):
    i = pl.program_id(0)
    x = x_ref[...]
    y = x + o_ref[...] @ wo[...] + bo[...]
    m = jnp.mean(y, axis=-1, keepdims=True)
    var = jnp.mean((y - m) ** 2, axis=-1, keepdims=True)
    h = (y - m) / jnp.sqrt(var + EPS) * g[...] + b[...]
    h_ref[...] = h
    l = h @ ws[...] + bs[...]
    lm = jnp.max(l, axis=-1, keepdims=True)
    el = jnp.exp(l - lm)
    z = jnp.sum(el, axis=-1, keepdims=True)
    prob = el / z
    pmax_ref[...] = 1.0 / z
    iota = lax.broadcasted_iota(jnp.int32, (BLK, E), 1)
    ridx = jnp.min(jnp.where(l == lm, iota, E), axis=-1, keepdims=True)
    routes_ref[...] = ridx

    @pl.when(i == 0)
    def _():
        stats_ref[...] = jnp.zeros_like(stats_ref)

    onehot = (iota == ridx).astype(jnp.float32)
    stats_ref[0:1, :] += jnp.sum(onehot, axis=0, keepdims=True)
    stats_ref[1:2, :] += jnp.sum(prob, axis=0, keepdims=True)


def _post(x, o, p):
    ts = pl.BlockSpec((BLK, D), lambda i: (i, 0))
    cs = pl.BlockSpec((BLK, 1), lambda i: (i, 0))
    return pl.pallas_call(
        _post_body,
        grid=(T // BLK,),
        in_specs=[ts, ts,
                  pl.BlockSpec((D, D), lambda i: (0, 0)),
                  pl.BlockSpec((1, D), lambda i: (0, 0)),
                  pl.BlockSpec((1, D), lambda i: (0, 0)),
                  pl.BlockSpec((1, D), lambda i: (0, 0)),
                  pl.BlockSpec((D, E), lambda i: (0, 0)),
                  pl.BlockSpec((1, E), lambda i: (0, 0))],
        out_specs=[ts, cs, cs, pl.BlockSpec((8, E), lambda i: (0, 0))],
        out_shape=[jax.ShapeDtypeStruct((T, D), jnp.float32),
                   jax.ShapeDtypeStruct((T, 1), jnp.int32),
                   jax.ShapeDtypeStruct((T, 1), jnp.float32),
                   jax.ShapeDtypeStruct((8, E), jnp.float32)],
    )(x, o, p['Wo'], p['bo'].reshape(1, D), p['g_mha'].reshape(1, D),
      p['b_mha'].reshape(1, D), p['Ws'], p['bs'].reshape(1, E))


def _shift_sum(x, sbuf):
    for sh in (1, 2, 4, 8):
        sbuf[pl.ds(16, 16)] = x
        x = x + sbuf[pl.ds(16 - sh, 16)]
    return x


def _spread_max(x, sbuf):
    for sh in (1, 2, 4, 8):
        sbuf[pl.ds(16, 16)] = x
        x = jnp.maximum(x, sbuf[pl.ds(16 + sh, 16)])
    return x


def _onehot_mask(rv, e):
    return 1 - jnp.minimum(rv ^ e, 1)


def _route_build_body(routes_hbm, h_hbm, pmax_hbm,
                      d_hbm, blk_hbm, hs_hbm, ps_hbm,
                      routes_v, routes64_v, grid_v, acc_v, d_v, blk_v,
                      sbuf, rows_v, pm_v, shared, sem):
    c = lax.axis_index("c")
    s = lax.axis_index("s")
    w = s * 2 + c
    iota = lax.iota(jnp.int32, 16)
    zeros = jnp.zeros((16,), jnp.int32)
    sbuf[pl.ds(0, 16)] = zeros
    sbuf[pl.ds(32, 16)] = zeros

    pltpu.sync_copy(routes_hbm.at[pl.ds(s * 128, 128)], routes_v)
    for half in range(2):
        for e in range(E):
            acc = zeros
            for vv in range(4):
                rv = routes_v[pl.ds(half * 64 + vv * 16, 16)]
                acc = acc + _onehot_mask(rv, e)
            acc_v[pl.ds(e * 16, 16)] = acc
        pltpu.sync_copy(acc_v, shared.at[pl.ds((s * 2 + half) * 128, 128)])
    plsc.subcore_barrier()
    pltpu.sync_copy(shared, grid_v)

    total = [jnp.zeros((16,), jnp.int32) for _ in range(E)]
    pre = [jnp.zeros((16,), jnp.int32) for _ in range(E)]
    for r in range(32):
        m_r = ((jnp.int32(r) - w) >> 31) & 1
        for e in range(E):
            acc = grid_v[pl.ds(r * 128 + e * 16, 16)]
            total[e] = total[e] + acc
            pre[e] = pre[e] + acc * m_r
    off_acc = jnp.zeros((16,), jnp.int32)
    offs = []
    incl = []
    for e in range(E):
        tot_e = _spread_max(_shift_sum(total[e], sbuf), sbuf)
        pre_e = _spread_max(_shift_sum(pre[e], sbuf), sbuf)
        pad_e = jnp.bitwise_and(tot_e + (BLK - 1), jnp.int32(-BLK))
        offs.append(off_acc + pre_e)
        off_acc = off_acc + pad_e
        incl.append(off_acc)

    @pl.when(w == 0)
    def _():
        for g2 in range(2):
            start = (iota + g2 * 16) * BLK
            eb = jnp.zeros((16,), jnp.int32)
            for e in range(E):
                eb = eb + 1 - (((start - incl[e]) >> 31) & 1)
            blk_v[pl.ds(g2 * 16, 16)] = jnp.minimum(eb, E - 1)
        pltpu.sync_copy(blk_v, blk_hbm)

    carry = [jnp.zeros((16,), jnp.int32) for _ in range(E)]
    pltpu.sync_copy(routes_hbm.at[pl.ds(w * 64, 64)], routes64_v)
    for vv in range(4):
        rv = routes64_v[pl.ds(vv * 16, 16)]
        d = jnp.zeros((16,), jnp.int32)
        for e in range(E):
            m = _onehot_mask(rv, e)
            cs = _shift_sum(m, sbuf)
            d = d + m * (offs[e] + carry[e] + cs - 1)
            carry[e] = carry[e] + _spread_max(cs, sbuf)
        d_v[pl.ds(vv * 16, 16)] = d
    pltpu.sync_copy(d_v, d_hbm.at[pl.ds(w * 64, 64)])

    pltpu.sync_copy(h_hbm.at[pl.ds(w * 64, 64)], rows_v)
    pltpu.async_copy(rows_v, hs_hbm.at[d_v], sem).wait()
    pltpu.sync_copy(pmax_hbm.at[pl.ds(w * 64, 64)], pm_v)
    pltpu.async_copy(pm_v, ps_hbm.at[d_v], sem).wait()


def _route_build(routes, h, pmax):
    mesh = plsc.VectorSubcoreMesh(core_axis_name="c", subcore_axis_name="s")
    f = pl.kernel(
        _route_build_body,
        out_type=[jax.ShapeDtypeStruct((T,), jnp.int32),
                  jax.ShapeDtypeStruct((NBPAD,), jnp.int32),
                  jax.ShapeDtypeStruct((NP, D), jnp.float32),
                  jax.ShapeDtypeStruct((NP,), jnp.float32)],
        mesh=mesh,
        scratch_types=[
            pltpu.VMEM((128,), jnp.int32),
            pltpu.VMEM((64,), jnp.int32),
            pltpu.VMEM((4096,), jnp.int32),
            pltpu.VMEM((128,), jnp.int32),
            pltpu.VMEM((64,), jnp.int32),
            pltpu.VMEM((32,), jnp.int32),
            pltpu.VMEM((48,), jnp.int32),
            pltpu.VMEM((64, D), jnp.float32),
            pltpu.VMEM((64,), jnp.float32),
            pltpu.VMEM_SHARED((4096,), jnp.int32),
            pltpu.SemaphoreType.DMA,
        ],
    )
    return f(routes, h, pmax)


def _ffn_body(eid_ref, hs_ref, ps_ref, w1_ref, b1_ref, w2_ref, b2_ref,
              ge_ref, be_ref, out_ref):
    x = hs_ref[...]
    a = jnp.maximum(lax.dot_general(
        x.astype(jnp.bfloat16), w1_ref[0], (((1,), (0,)), ((), ())),
        preferred_element_type=jnp.float32) + b1_ref[0], 0.0)
    o = lax.dot_general(
        a.astype(jnp.bfloat16), w2_ref[0], (((1,), (0,)), ((), ())),
        preferred_element_type=jnp.float32) + b2_ref[0]
    y = x + o
    m = jnp.mean(y, axis=-1, keepdims=True)
    var = jnp.mean((y - m) ** 2, axis=-1, keepdims=True)
    hn = (y - m) / jnp.sqrt(var + EPS) * ge_ref[0] + be_ref[0]
    out_ref[...] = hn * ps_ref[...]


def _ffn(blk_expert, hs, ps2, p):
    grid_spec = pltpu.PrefetchScalarGridSpec(
        num_scalar_prefetch=1,
        grid=(NB,),
        in_specs=[
            pl.BlockSpec((BLK, D), lambda b, eid: (b, 0)),
            pl.BlockSpec((BLK, 1), lambda b, eid: (b, 0)),
            pl.BlockSpec((1, D, DFF), lambda b, eid: (eid[b], 0, 0)),
            pl.BlockSpec((1, 1, DFF), lambda b, eid: (eid[b], 0, 0)),
            pl.BlockSpec((1, DFF, D), lambda b, eid: (eid[b], 0, 0)),
            pl.BlockSpec((1, 1, D), lambda b, eid: (eid[b], 0, 0)),
            pl.BlockSpec((1, 1, D), lambda b, eid: (eid[b], 0, 0)),
            pl.BlockSpec((1, 1, D), lambda b, eid: (eid[b], 0, 0)),
        ],
        out_specs=pl.BlockSpec((BLK, D), lambda b, eid: (b, 0)),
    )
    return pl.pallas_call(
        _ffn_body,
        grid_spec=grid_spec,
        out_shape=jax.ShapeDtypeStruct((NP, D), jnp.float32),
    )(blk_expert, hs, ps2, p['W1'].astype(jnp.bfloat16),
      p['b1'].reshape(E, 1, DFF),
      p['W2'].astype(jnp.bfloat16), p['b2'].reshape(E, 1, D),
      p['ge'].reshape(E, 1, D), p['be'].reshape(E, 1, D))


def _unperm_body(d_hbm, os_hbm, out_hbm, d_v, rows_v, sem):
    c = lax.axis_index("c")
    s = lax.axis_index("s")
    w = s * 2 + c
    pltpu.sync_copy(d_hbm.at[pl.ds(w * 64, 64)], d_v)
    pltpu.async_copy(os_hbm.at[d_v], rows_v, sem).wait()
    pltpu.sync_copy(rows_v, out_hbm.at[pl.ds(w * 64, 64)])


def _unperm(d, out_sorted):
    mesh = plsc.VectorSubcoreMesh(core_axis_name="c", subcore_axis_name="s")
    f = pl.kernel(
        _unperm_body,
        out_type=jax.ShapeDtypeStruct((T, D), jnp.float32),
        mesh=mesh,
        scratch_types=[
            pltpu.VMEM((64,), jnp.int32),
            pltpu.VMEM((64, D), jnp.float32),
            pltpu.SemaphoreType.DMA,
        ],
    )
    return f(d, out_sorted)


def kernel(x, params):
    p = params
    q, k, v = _proj(x, p)
    to_heads = lambda a: a.reshape(T, H, DH).transpose(1, 0, 2)
    o3 = _attn(to_heads(q), to_heads(k), to_heads(v))
    o = o3.transpose(1, 0, 2).reshape(T, D)
    h, routes2d, pmax2d, stats = _post(x, o, p)
    routes = routes2d.reshape(T)
    pmax = pmax2d.reshape(T)
    d, blk_expert, hs, ps = _route_build(routes, h, pmax)
    out_sorted = _ffn(blk_expert, hs, ps.reshape(NP, 1), p)
    out = _unperm(d, out_sorted)
    return out, stats[0], stats[1], jnp.int32(0), pmax


# speedup vs baseline: 1.3717x; 1.3717x over previous
import functools

import jax
import jax.numpy as jnp
from jax import lax
from jax.experimental import pallas as pl
from jax.experimental.pallas import tpu as pltpu
from jax.experimental.pallas import tpu_sc as plsc

D = 768
H = 12
DH = D // H
E = 8
DFF = 3072
T = 2048
EPS = 1e-5
BLK = 128
NP = T + E * BLK
NB = NP // BLK
NBPAD = 32
TB = 256
RB = 1024
SCALE = 1.0 / (DH ** 0.5)


def _proj_body(x_ref, wk, bk, wq, bq, wv, bv, wqi, bqi, wki, bki, wvi, bvi,
               q_ref, k_ref, v_ref):
    x = x_ref[...]
    q_ref[...] = (x @ wk[...] + bk[...]) @ wqi[...] + bqi[...]
    k_ref[...] = (x @ wq[...] + bq[...]) @ wki[...] + bki[...]
    v_ref[...] = (x @ wv[...] + bv[...]) @ wvi[...] + bvi[...]


def _proj(x, p):
    w_spec = pl.BlockSpec((D, D), lambda i: (0, 0))
    b_spec = pl.BlockSpec((1, D), lambda i: (0, 0))
    ts = pl.BlockSpec((TB, D), lambda i: (i, 0))
    out = jax.ShapeDtypeStruct((T, D), jnp.float32)
    return pl.pallas_call(
        _proj_body,
        grid=(T // TB,),
        in_specs=[ts, w_spec, b_spec, w_spec, b_spec, w_spec, b_spec,
                  w_spec, b_spec, w_spec, b_spec, w_spec, b_spec],
        out_specs=[ts, ts, ts],
        out_shape=[out, out, out],
    )(x, p['Wk'], p['bk'].reshape(1, D), p['Wq'], p['bq'].reshape(1, D),
      p['Wv'], p['bv'].reshape(1, D), p['Wqi'], p['bqi'].reshape(1, D),
      p['Wki'], p['bki'].reshape(1, D), p['Wvi'], p['bvi'].reshape(1, D))


def _attn_body(q_ref, k_ref, v_ref, o_ref):
    qp = q_ref[...] * SCALE
    outs = []
    for j in range(2):
        q = qp[:, j * DH:(j + 1) * DH]
        k = k_ref[:, j * DH:(j + 1) * DH]
        v = v_ref[:, j * DH:(j + 1) * DH]
        s = lax.dot_general(q, k, (((1,), (1,)), ((), ())),
                            preferred_element_type=jnp.float32)
        m = jnp.max(s, axis=-1, keepdims=True)
        e = jnp.exp(s - m)
        z = jnp.sum(e, axis=-1, keepdims=True)
        o = lax.dot_general(e, v, (((1,), (0,)), ((), ())),
                            preferred_element_type=jnp.float32)
        outs.append(o / z)
    o_ref[...] = jnp.concatenate(outs, axis=1)


def _attn(q, k, v):
    return pl.pallas_call(
        _attn_body,
        grid=(T // RB, H // 2),
        in_specs=[pl.BlockSpec((RB, 2 * DH), lambda i, h2: (i, h2)),
                  pl.BlockSpec((T, 2 * DH), lambda i, h2: (0, h2)),
                  pl.BlockSpec((T, 2 * DH), lambda i, h2: (0, h2))],
        out_specs=pl.BlockSpec((RB, 2 * DH), lambda i, h2: (i, h2)),
        out_shape=jax.ShapeDtypeStruct((T, D), jnp.float32),
    )(q, k, v)


def _post_body(x_ref, o_ref, wo, bo, g, b, ws, bs,
               h_ref, routes_ref, pmax_ref, stats_ref):
    i = pl.program_id(0)
    x = x_ref[...]
    y = x + o_ref[...] @ wo[...] + bo[...]
    m = jnp.mean(y, axis=-1, keepdims=True)
    var = jnp.mean((y - m) ** 2, axis=-1, keepdims=True)
    h = (y - m) / jnp.sqrt(var + EPS) * g[...] + b[...]
    h_ref[...] = h
    l = h @ ws[...] + bs[...]
    lm = jnp.max(l, axis=-1, keepdims=True)
    el = jnp.exp(l - lm)
    z = jnp.sum(el, axis=-1, keepdims=True)
    prob = el / z
    pmax_ref[...] = 1.0 / z
    iota = lax.broadcasted_iota(jnp.int32, (BLK, E), 1)
    ridx = jnp.min(jnp.where(l == lm, iota, E), axis=-1, keepdims=True)
    routes_ref[...] = ridx

    @pl.when(i == 0)
    def _():
        stats_ref[...] = jnp.zeros_like(stats_ref)

    onehot = (iota == ridx).astype(jnp.float32)
    stats_ref[0:1, :] += jnp.sum(onehot, axis=0, keepdims=True)
    stats_ref[1:2, :] += jnp.sum(prob, axis=0, keepdims=True)


def _post(x, o, p):
    ts = pl.BlockSpec((BLK, D), lambda i: (i, 0))
    cs = pl.BlockSpec((BLK, 1), lambda i: (i, 0))
    return pl.pallas_call(
        _post_body,
        grid=(T // BLK,),
        in_specs=[ts, ts,
                  pl.BlockSpec((D, D), lambda i: (0, 0)),
                  pl.BlockSpec((1, D), lambda i: (0, 0)),
                  pl.BlockSpec((1, D), lambda i: (0, 0)),
                  pl.BlockSpec((1, D), lambda i: (0, 0)),
                  pl.BlockSpec((D, E), lambda i: (0, 0)),
                  pl.BlockSpec((1, E), lambda i: (0, 0))],
        out_specs=[ts, cs, cs, pl.BlockSpec((8, E), lambda i: (0, 0))],
        out_shape=[jax.ShapeDtypeStruct((T, D), jnp.float32),
                   jax.ShapeDtypeStruct((T, 1), jnp.int32),
                   jax.ShapeDtypeStruct((T, 1), jnp.float32),
                   jax.ShapeDtypeStruct((8, E), jnp.float32)],
    )(x, o, p['Wo'], p['bo'].reshape(1, D), p['g_mha'].reshape(1, D),
      p['b_mha'].reshape(1, D), p['Ws'], p['bs'].reshape(1, E))


def _shift_sum(x, sbuf):
    for sh in (1, 2, 4, 8):
        sbuf[pl.ds(16, 16)] = x
        x = x + sbuf[pl.ds(16 - sh, 16)]
    return x


def _spread_max(x, sbuf):
    for sh in (1, 2, 4, 8):
        sbuf[pl.ds(16, 16)] = x
        x = jnp.maximum(x, sbuf[pl.ds(16 + sh, 16)])
    return x


def _onehot_mask(rv, e):
    return 1 - jnp.minimum(rv ^ e, 1)


def _route_build_body(routes_hbm, h_hbm, pmax_hbm,
                      d_hbm, blk_hbm, hs_hbm, ps_hbm,
                      routes_v, routes64_v, grid_v, acc_v, d_v, blk_v,
                      sbuf, rows_v, pm_v, shared, sem):
    c = lax.axis_index("c")
    s = lax.axis_index("s")
    w = s * 2 + c
    iota = lax.iota(jnp.int32, 16)
    zeros = jnp.zeros((16,), jnp.int32)
    sbuf[pl.ds(0, 16)] = zeros
    sbuf[pl.ds(32, 16)] = zeros

    pltpu.sync_copy(routes_hbm.at[pl.ds(s * 128, 128)], routes_v)
    for half in range(2):
        for e in range(E):
            acc = zeros
            for vv in range(4):
                rv = routes_v[pl.ds(half * 64 + vv * 16, 16)]
                acc = acc + _onehot_mask(rv, e)
            acc_v[pl.ds(e * 16, 16)] = acc
        pltpu.sync_copy(acc_v, shared.at[pl.ds((s * 2 + half) * 128, 128)])
    plsc.subcore_barrier()
    pltpu.sync_copy(shared, grid_v)

    total = [jnp.zeros((16,), jnp.int32) for _ in range(E)]
    pre = [jnp.zeros((16,), jnp.int32) for _ in range(E)]
    for r in range(32):
        m_r = ((jnp.int32(r) - w) >> 31) & 1
        for e in range(E):
            acc = grid_v[pl.ds(r * 128 + e * 16, 16)]
            total[e] = total[e] + acc
            pre[e] = pre[e] + acc * m_r
    off_acc = jnp.zeros((16,), jnp.int32)
    offs = []
    incl = []
    for e in range(E):
        tot_e = _spread_max(_shift_sum(total[e], sbuf), sbuf)
        pre_e = _spread_max(_shift_sum(pre[e], sbuf), sbuf)
        pad_e = jnp.bitwise_and(tot_e + (BLK - 1), jnp.int32(-BLK))
        offs.append(off_acc + pre_e)
        off_acc = off_acc + pad_e
        incl.append(off_acc)

    @pl.when(w == 0)
    def _():
        for g2 in range(2):
            start = (iota + g2 * 16) * BLK
            eb = jnp.zeros((16,), jnp.int32)
            for e in range(E):
                eb = eb + 1 - (((start - incl[e]) >> 31) & 1)
            blk_v[pl.ds(g2 * 16, 16)] = jnp.minimum(eb, E - 1)
        pltpu.sync_copy(blk_v, blk_hbm)

    carry = [jnp.zeros((16,), jnp.int32) for _ in range(E)]
    pltpu.sync_copy(routes_hbm.at[pl.ds(w * 64, 64)], routes64_v)
    for vv in range(4):
        rv = routes64_v[pl.ds(vv * 16, 16)]
        d = jnp.zeros((16,), jnp.int32)
        for e in range(E):
            m = _onehot_mask(rv, e)
            cs = _shift_sum(m, sbuf)
            d = d + m * (offs[e] + carry[e] + cs - 1)
            carry[e] = carry[e] + _spread_max(cs, sbuf)
        d_v[pl.ds(vv * 16, 16)] = d
    pltpu.sync_copy(d_v, d_hbm.at[pl.ds(w * 64, 64)])

    pltpu.sync_copy(h_hbm.at[pl.ds(w * 64, 64)], rows_v)
    pltpu.async_copy(rows_v, hs_hbm.at[d_v], sem).wait()
    pltpu.sync_copy(pmax_hbm.at[pl.ds(w * 64, 64)], pm_v)
    pltpu.async_copy(pm_v, ps_hbm.at[d_v], sem).wait()


def _route_build(routes, h, pmax):
    mesh = plsc.VectorSubcoreMesh(core_axis_name="c", subcore_axis_name="s")
    f = pl.kernel(
        _route_build_body,
        out_type=[jax.ShapeDtypeStruct((T,), jnp.int32),
                  jax.ShapeDtypeStruct((NBPAD,), jnp.int32),
                  jax.ShapeDtypeStruct((NP, D), jnp.float32),
                  jax.ShapeDtypeStruct((NP,), jnp.float32)],
        mesh=mesh,
        scratch_types=[
            pltpu.VMEM((128,), jnp.int32),
            pltpu.VMEM((64,), jnp.int32),
            pltpu.VMEM((4096,), jnp.int32),
            pltpu.VMEM((128,), jnp.int32),
            pltpu.VMEM((64,), jnp.int32),
            pltpu.VMEM((32,), jnp.int32),
            pltpu.VMEM((48,), jnp.int32),
            pltpu.VMEM((64, D), jnp.float32),
            pltpu.VMEM((64,), jnp.float32),
            pltpu.VMEM_SHARED((4096,), jnp.int32),
            pltpu.SemaphoreType.DMA,
        ],
    )
    return f(routes, h, pmax)


def _ffn_body(eid_ref, hs_ref, ps_ref, w1_ref, b1_ref, w2_ref, b2_ref,
              ge_ref, be_ref, out_ref):
    x = hs_ref[...]
    a = jnp.maximum(x @ w1_ref[0] + b1_ref[0], 0.0)
    o = a @ w2_ref[0] + b2_ref[0]
    y = x + o
    m = jnp.mean(y, axis=-1, keepdims=True)
    var = jnp.mean((y - m) ** 2, axis=-1, keepdims=True)
    hn = (y - m) / jnp.sqrt(var + EPS) * ge_ref[0] + be_ref[0]
    out_ref[...] = hn * ps_ref[...]


def _ffn(blk_expert, hs, ps2, p):
    grid_spec = pltpu.PrefetchScalarGridSpec(
        num_scalar_prefetch=1,
        grid=(NB,),
        in_specs=[
            pl.BlockSpec((BLK, D), lambda b, eid: (b, 0)),
            pl.BlockSpec((BLK, 1), lambda b, eid: (b, 0)),
            pl.BlockSpec((1, D, DFF), lambda b, eid: (eid[b], 0, 0)),
            pl.BlockSpec((1, 1, DFF), lambda b, eid: (eid[b], 0, 0)),
            pl.BlockSpec((1, DFF, D), lambda b, eid: (eid[b], 0, 0)),
            pl.BlockSpec((1, 1, D), lambda b, eid: (eid[b], 0, 0)),
            pl.BlockSpec((1, 1, D), lambda b, eid: (eid[b], 0, 0)),
            pl.BlockSpec((1, 1, D), lambda b, eid: (eid[b], 0, 0)),
        ],
        out_specs=pl.BlockSpec((BLK, D), lambda b, eid: (b, 0)),
    )
    return pl.pallas_call(
        _ffn_body,
        grid_spec=grid_spec,
        out_shape=jax.ShapeDtypeStruct((NP, D), jnp.float32),
    )(blk_expert, hs, ps2, p['W1'], p['b1'].reshape(E, 1, DFF),
      p['W2'], p['b2'].reshape(E, 1, D),
      p['ge'].reshape(E, 1, D), p['be'].reshape(E, 1, D))


def _unperm_body(d_hbm, os_hbm, out_hbm, d_v, rows_v, sem):
    c = lax.axis_index("c")
    s = lax.axis_index("s")
    w = s * 2 + c
    pltpu.sync_copy(d_hbm.at[pl.ds(w * 64, 64)], d_v)
    pltpu.async_copy(os_hbm.at[d_v], rows_v, sem).wait()
    pltpu.sync_copy(rows_v, out_hbm.at[pl.ds(w * 64, 64)])


def _unperm(d, out_sorted):
    mesh = plsc.VectorSubcoreMesh(core_axis_name="c", subcore_axis_name="s")
    f = pl.kernel(
        _unperm_body,
        out_type=jax.ShapeDtypeStruct((T, D), jnp.float32),
        mesh=mesh,
        scratch_types=[
            pltpu.VMEM((64,), jnp.int32),
            pltpu.VMEM((64, D), jnp.float32),
            pltpu.SemaphoreType.DMA,
        ],
    )
    return f(d, out_sorted)


def kernel(x, params):
    p = params
    q, k, v = _proj(x, p)
    o = _attn(q, k, v)
    h, routes2d, pmax2d, stats = _post(x, o, p)
    routes = routes2d.reshape(T)
    pmax = pmax2d.reshape(T)
    d, blk_expert, hs, ps = _route_build(routes, h, pmax)
    out_sorted = _ffn(blk_expert, hs, ps.reshape(NP, 1), p)
    out = _unperm(d, out_sorted)
    return out, stats[0], stats[1], jnp.int32(0), pmax


# speedup vs baseline: 1.4696x; 1.0714x over previous
import functools

import jax
import jax.numpy as jnp
from jax import lax
from jax.experimental import pallas as pl
from jax.experimental.pallas import tpu as pltpu
from jax.experimental.pallas import tpu_sc as plsc

D = 768
H = 12
DH = D // H
E = 8
DFF = 3072
T = 2048
EPS = 1e-5
BLK = 128
NP = T + E * BLK
NB = NP // BLK
NBPAD = 32
TB = 256
RB = 1024
SCALE = 1.0 / (DH ** 0.5)


def _proj_body(x_ref, wk, bk, wq, bq, wv, bv, wqi, bqi, wki, bki, wvi, bvi,
               q_ref, k_ref, v_ref):
    x = x_ref[...]
    q_ref[...] = (x @ wk[...] + bk[...]) @ wqi[...] + bqi[...]
    k_ref[...] = (x @ wq[...] + bq[...]) @ wki[...] + bki[...]
    v_ref[...] = (x @ wv[...] + bv[...]) @ wvi[...] + bvi[...]


def _proj(x, p):
    w_spec = pl.BlockSpec((D, D), lambda i: (0, 0))
    b_spec = pl.BlockSpec((1, D), lambda i: (0, 0))
    ts = pl.BlockSpec((TB, D), lambda i: (i, 0))
    out = jax.ShapeDtypeStruct((T, D), jnp.float32)
    return pl.pallas_call(
        _proj_body,
        grid=(T // TB,),
        in_specs=[ts, w_spec, b_spec, w_spec, b_spec, w_spec, b_spec,
                  w_spec, b_spec, w_spec, b_spec, w_spec, b_spec],
        out_specs=[ts, ts, ts],
        out_shape=[out, out, out],
    )(x, p['Wk'], p['bk'].reshape(1, D), p['Wq'], p['bq'].reshape(1, D),
      p['Wv'], p['bv'].reshape(1, D), p['Wqi'], p['bqi'].reshape(1, D),
      p['Wki'], p['bki'].reshape(1, D), p['Wvi'], p['bvi'].reshape(1, D))


H2 = H // 2


def _attn_body(x_ref, q_ref, k_ref, v_ref, wo, bo, g, b, ws, bs,
               h_ref, routes_ref, pmax_ref, stats_ref, o_scr):
    i = pl.program_id(0)
    h2 = pl.program_id(1)

    @pl.when(h2 < H2)
    def _():
        qp = q_ref[...] * SCALE
        outs = []
        for j in range(2):
            q = qp[:, j * DH:(j + 1) * DH]
            k = k_ref[:, j * DH:(j + 1) * DH]
            v = v_ref[:, j * DH:(j + 1) * DH]
            s = lax.dot_general(q, k, (((1,), (1,)), ((), ())),
                                preferred_element_type=jnp.float32)
            m = jnp.max(s, axis=-1, keepdims=True)
            e = jnp.exp(s - m)
            z = jnp.sum(e, axis=-1, keepdims=True)
            o = lax.dot_general(e, v, (((1,), (0,)), ((), ())),
                                preferred_element_type=jnp.float32)
            outs.append(o / z)
        o_scr[:, pl.ds(pl.multiple_of(h2 * 2 * DH, 128), 2 * DH)] = (
            jnp.concatenate(outs, axis=1))

    @pl.when(h2 == H2)
    def _():
        x = x_ref[...]
        y = x + o_scr[...] @ wo[...] + bo[...]
        m = jnp.mean(y, axis=-1, keepdims=True)
        var = jnp.mean((y - m) ** 2, axis=-1, keepdims=True)
        hh = (y - m) / jnp.sqrt(var + EPS) * g[...] + b[...]
        h_ref[...] = hh
        l = hh @ ws[...] + bs[...]
        lm = jnp.max(l, axis=-1, keepdims=True)
        el = jnp.exp(l - lm)
        z = jnp.sum(el, axis=-1, keepdims=True)
        prob = el / z
        pmax_ref[...] = 1.0 / z
        iota = lax.broadcasted_iota(jnp.int32, (RB, E), 1)
        ridx = jnp.min(jnp.where(l == lm, iota, E), axis=-1, keepdims=True)
        routes_ref[...] = ridx

        @pl.when(i == 0)
        def _():
            stats_ref[...] = jnp.zeros_like(stats_ref)

        onehot = (iota == ridx).astype(jnp.float32)
        stats_ref[0:1, :] += jnp.sum(onehot, axis=0, keepdims=True)
        stats_ref[1:2, :] += jnp.sum(prob, axis=0, keepdims=True)


def _attn(x, q, k, v, p):
    c = pl.BlockSpec((RB, 1), lambda i, h2: (i, 0))
    hp = lambda i, h2: (i, jnp.minimum(h2, H2 - 1))
    kp = lambda i, h2: (0, jnp.minimum(h2, H2 - 1))
    z = lambda i, h2: (0, 0)
    return pl.pallas_call(
        _attn_body,
        grid=(T // RB, H2 + 1),
        in_specs=[pl.BlockSpec((RB, D), lambda i, h2: (i, 0)),
                  pl.BlockSpec((RB, 2 * DH), hp),
                  pl.BlockSpec((T, 2 * DH), kp),
                  pl.BlockSpec((T, 2 * DH), kp),
                  pl.BlockSpec((D, D), z),
                  pl.BlockSpec((1, D), z),
                  pl.BlockSpec((1, D), z),
                  pl.BlockSpec((1, D), z),
                  pl.BlockSpec((D, E), z),
                  pl.BlockSpec((1, E), z)],
        out_specs=[pl.BlockSpec((RB, D), lambda i, h2: (i, 0)),
                   c, c, pl.BlockSpec((8, E), z)],
        out_shape=[jax.ShapeDtypeStruct((T, D), jnp.float32),
                   jax.ShapeDtypeStruct((T, 1), jnp.int32),
                   jax.ShapeDtypeStruct((T, 1), jnp.float32),
                   jax.ShapeDtypeStruct((8, E), jnp.float32)],
        scratch_shapes=[pltpu.VMEM((RB, D), jnp.float32)],
    )(x, q, k, v, p['Wo'], p['bo'].reshape(1, D), p['g_mha'].reshape(1, D),
      p['b_mha'].reshape(1, D), p['Ws'], p['bs'].reshape(1, E))


def _shift_sum(x, sbuf):
    for sh in (1, 2, 4, 8):
        sbuf[pl.ds(16, 16)] = x
        x = x + sbuf[pl.ds(16 - sh, 16)]
    return x


def _spread_max(x, sbuf):
    for sh in (1, 2, 4, 8):
        sbuf[pl.ds(16, 16)] = x
        x = jnp.maximum(x, sbuf[pl.ds(16 + sh, 16)])
    return x


def _onehot_mask(rv, e):
    return 1 - jnp.minimum(rv ^ e, 1)


def _route_build_body(routes_hbm, h_hbm, pmax_hbm,
                      d_hbm, blk_hbm, hs_hbm, ps_hbm,
                      routes_v, routes64_v, grid_v, acc_v, d_v, blk_v,
                      sbuf, rows_v, pm_v, shared, sem, sem2):
    c = lax.axis_index("c")
    s = lax.axis_index("s")
    w = s * 2 + c
    iota = lax.iota(jnp.int32, 16)
    zeros = jnp.zeros((16,), jnp.int32)
    sbuf[pl.ds(0, 16)] = zeros
    sbuf[pl.ds(32, 16)] = zeros

    cp_rows = pltpu.async_copy(h_hbm.at[pl.ds(w * 64, 64)], rows_v, sem)
    cp_pm = pltpu.async_copy(pmax_hbm.at[pl.ds(w * 64, 64)], pm_v, sem2)

    pltpu.sync_copy(routes_hbm.at[pl.ds(s * 128, 128)], routes_v)
    for half in range(2):
        for e in range(E):
            acc = zeros
            for vv in range(4):
                rv = routes_v[pl.ds(half * 64 + vv * 16, 16)]
                acc = acc + _onehot_mask(rv, e)
            acc_v[pl.ds(e * 16, 16)] = acc
        pltpu.sync_copy(acc_v, shared.at[pl.ds((s * 2 + half) * 128, 128)])
    plsc.subcore_barrier()
    pltpu.sync_copy(shared, grid_v)

    total = [jnp.zeros((16,), jnp.int32) for _ in range(E)]
    pre = [jnp.zeros((16,), jnp.int32) for _ in range(E)]
    for r in range(32):
        m_r = ((jnp.int32(r) - w) >> 31) & 1
        for e in range(E):
            acc = grid_v[pl.ds(r * 128 + e * 16, 16)]
            total[e] = total[e] + acc
            pre[e] = pre[e] + acc * m_r
    off_acc = jnp.zeros((16,), jnp.int32)
    offs = []
    incl = []
    for e in range(E):
        tot_e = _spread_max(_shift_sum(total[e], sbuf), sbuf)
        pre_e = _spread_max(_shift_sum(pre[e], sbuf), sbuf)
        pad_e = jnp.bitwise_and(tot_e + (BLK - 1), jnp.int32(-BLK))
        offs.append(off_acc + pre_e)
        off_acc = off_acc + pad_e
        incl.append(off_acc)

    @pl.when(w == 0)
    def _():
        for g2 in range(2):
            start = (iota + g2 * 16) * BLK
            eb = jnp.zeros((16,), jnp.int32)
            for e in range(E):
                eb = eb + 1 - (((start - incl[e]) >> 31) & 1)
            blk_v[pl.ds(g2 * 16, 16)] = jnp.minimum(eb, E - 1)
        pltpu.sync_copy(blk_v, blk_hbm)

    carry = [jnp.zeros((16,), jnp.int32) for _ in range(E)]
    pltpu.sync_copy(routes_hbm.at[pl.ds(w * 64, 64)], routes64_v)
    for vv in range(4):
        rv = routes64_v[pl.ds(vv * 16, 16)]
        d = jnp.zeros((16,), jnp.int32)
        for e in range(E):
            m = _onehot_mask(rv, e)
            cs = _shift_sum(m, sbuf)
            d = d + m * (offs[e] + carry[e] + cs - 1)
            carry[e] = carry[e] + _spread_max(cs, sbuf)
        d_v[pl.ds(vv * 16, 16)] = d
    pltpu.sync_copy(d_v, d_hbm.at[pl.ds(w * 64, 64)])

    cp_rows.wait()
    cp_pm.wait()
    pltpu.async_copy(rows_v, hs_hbm.at[d_v], sem).wait()
    pltpu.async_copy(pm_v, ps_hbm.at[d_v], sem2).wait()


def _route_build(routes, h, pmax):
    mesh = plsc.VectorSubcoreMesh(core_axis_name="c", subcore_axis_name="s")
    f = pl.kernel(
        _route_build_body,
        out_type=[jax.ShapeDtypeStruct((T,), jnp.int32),
                  jax.ShapeDtypeStruct((NBPAD,), jnp.int32),
                  jax.ShapeDtypeStruct((NP, D), jnp.float32),
                  jax.ShapeDtypeStruct((NP,), jnp.float32)],
        mesh=mesh,
        scratch_types=[
            pltpu.VMEM((128,), jnp.int32),
            pltpu.VMEM((64,), jnp.int32),
            pltpu.VMEM((4096,), jnp.int32),
            pltpu.VMEM((128,), jnp.int32),
            pltpu.VMEM((64,), jnp.int32),
            pltpu.VMEM((32,), jnp.int32),
            pltpu.VMEM((48,), jnp.int32),
            pltpu.VMEM((64, D), jnp.float32),
            pltpu.VMEM((64,), jnp.float32),
            pltpu.VMEM_SHARED((4096,), jnp.int32),
            pltpu.SemaphoreType.DMA,
            pltpu.SemaphoreType.DMA,
        ],
    )
    return f(routes, h, pmax)


def _ffn_body(eid_ref, hs_ref, ps_ref, w1_ref, b1_ref, w2_ref, b2_ref,
              ge_ref, be_ref, out_ref):
    x = hs_ref[...]
    a = jnp.maximum(x @ w1_ref[0] + b1_ref[0], 0.0)
    o = a @ w2_ref[0] + b2_ref[0]
    y = x + o
    m = jnp.mean(y, axis=-1, keepdims=True)
    var = jnp.mean((y - m) ** 2, axis=-1, keepdims=True)
    hn = (y - m) / jnp.sqrt(var + EPS) * ge_ref[0] + be_ref[0]
    out_ref[...] = hn * ps_ref[...]


def _ffn(blk_expert, hs, ps2, p):
    grid_spec = pltpu.PrefetchScalarGridSpec(
        num_scalar_prefetch=1,
        grid=(NB,),
        in_specs=[
            pl.BlockSpec((BLK, D), lambda b, eid: (b, 0)),
            pl.BlockSpec((BLK, 1), lambda b, eid: (b, 0)),
            pl.BlockSpec((1, D, DFF), lambda b, eid: (eid[b], 0, 0)),
            pl.BlockSpec((1, 1, DFF), lambda b, eid: (eid[b], 0, 0)),
            pl.BlockSpec((1, DFF, D), lambda b, eid: (eid[b], 0, 0)),
            pl.BlockSpec((1, 1, D), lambda b, eid: (eid[b], 0, 0)),
            pl.BlockSpec((1, 1, D), lambda b, eid: (eid[b], 0, 0)),
            pl.BlockSpec((1, 1, D), lambda b, eid: (eid[b], 0, 0)),
        ],
        out_specs=pl.BlockSpec((BLK, D), lambda b, eid: (b, 0)),
    )
    return pl.pallas_call(
        _ffn_body,
        grid_spec=grid_spec,
        out_shape=jax.ShapeDtypeStruct((NP, D), jnp.float32),
    )(blk_expert, hs, ps2, p['W1'], p['b1'].reshape(E, 1, DFF),
      p['W2'], p['b2'].reshape(E, 1, D),
      p['ge'].reshape(E, 1, D), p['be'].reshape(E, 1, D))


def _unperm_body(d_hbm, os_hbm, out_hbm, d_v, rows_v, sem):
    c = lax.axis_index("c")
    s = lax.axis_index("s")
    w = s * 2 + c
    pltpu.sync_copy(d_hbm.at[pl.ds(w * 64, 64)], d_v)
    pltpu.async_copy(os_hbm.at[d_v], rows_v, sem).wait()
    pltpu.sync_copy(rows_v, out_hbm.at[pl.ds(w * 64, 64)])


def _unperm(d, out_sorted):
    mesh = plsc.VectorSubcoreMesh(core_axis_name="c", subcore_axis_name="s")
    f = pl.kernel(
        _unperm_body,
        out_type=jax.ShapeDtypeStruct((T, D), jnp.float32),
        mesh=mesh,
        scratch_types=[
            pltpu.VMEM((64,), jnp.int32),
            pltpu.VMEM((64, D), jnp.float32),
            pltpu.SemaphoreType.DMA,
        ],
    )
    return f(d, out_sorted)


def kernel(x, params):
    p = params
    q, k, v = _proj(x, p)
    h, routes2d, pmax2d, stats = _attn(x, q, k, v, p)
    routes = routes2d.reshape(T)
    pmax = pmax2d.reshape(T)
    d, blk_expert, hs, ps = _route_build(routes, h, pmax)
    out_sorted = _ffn(blk_expert, hs, ps.reshape(NP, 1), p)
    out = _unperm(d, out_sorted)
    return out, stats[0], stats[1], jnp.int32(0), pmax


# speedup vs baseline: 1.5986x; 1.0878x over previous
import functools

import jax
import jax.numpy as jnp
from jax import lax
from jax.experimental import pallas as pl
from jax.experimental.pallas import tpu as pltpu
from jax.experimental.pallas import tpu_sc as plsc

D = 768
H = 12
DH = D // H
E = 8
DFF = 3072
T = 2048
EPS = 1e-5
BLK = 128
NP = T + E * BLK
NB = NP // BLK
NBPAD = 32
TB = 256
RB = 1024
SCALE = 1.0 / (DH ** 0.5)


def _proj_body(x_ref, wk, bk, wq, bq, wv, bv, wqi, bqi, wki, bki, wvi, bvi,
               q_ref, k_ref, v_ref):
    x = x_ref[...]
    q_ref[...] = (x @ wk[...] + bk[...]) @ wqi[...] + bqi[...]
    k_ref[...] = (x @ wq[...] + bq[...]) @ wki[...] + bki[...]
    v_ref[...] = (x @ wv[...] + bv[...]) @ wvi[...] + bvi[...]


def _proj(x, p):
    w_spec = pl.BlockSpec((D, D), lambda i: (0, 0))
    b_spec = pl.BlockSpec((1, D), lambda i: (0, 0))
    ts = pl.BlockSpec((TB, D), lambda i: (i, 0))
    out = jax.ShapeDtypeStruct((T, D), jnp.float32)
    return pl.pallas_call(
        _proj_body,
        grid=(T // TB,),
        in_specs=[ts, w_spec, b_spec, w_spec, b_spec, w_spec, b_spec,
                  w_spec, b_spec, w_spec, b_spec, w_spec, b_spec],
        out_specs=[ts, ts, ts],
        out_shape=[out, out, out],
    )(x, p['Wk'], p['bk'].reshape(1, D), p['Wq'], p['bq'].reshape(1, D),
      p['Wv'], p['bv'].reshape(1, D), p['Wqi'], p['bqi'].reshape(1, D),
      p['Wki'], p['bki'].reshape(1, D), p['Wvi'], p['bvi'].reshape(1, D))


H2 = H // 2


def _attn_body(x_ref, q_ref, k_ref, v_ref, wo, bo, g, b, ws, bs,
               h_ref, routes_ref, pmax_ref, stats_ref, o_scr):
    i = pl.program_id(0)
    h2 = pl.program_id(1)

    @pl.when(h2 < H2)
    def _():
        qp = q_ref[...] * SCALE
        outs = []
        for j in range(2):
            q = qp[:, j * DH:(j + 1) * DH]
            k = k_ref[:, j * DH:(j + 1) * DH]
            v = v_ref[:, j * DH:(j + 1) * DH]
            s = lax.dot_general(q, k, (((1,), (1,)), ((), ())),
                                preferred_element_type=jnp.float32)
            e = jnp.exp(s)
            z = jnp.sum(e, axis=-1, keepdims=True)
            o = lax.dot_general(e, v, (((1,), (0,)), ((), ())),
                                preferred_element_type=jnp.float32)
            outs.append(o / z)
        o_scr[:, pl.ds(pl.multiple_of(h2 * 2 * DH, 128), 2 * DH)] = (
            jnp.concatenate(outs, axis=1))

    @pl.when(h2 == H2)
    def _():
        x = x_ref[...]
        y = x + o_scr[...] @ wo[...] + bo[...]
        m = jnp.mean(y, axis=-1, keepdims=True)
        var = jnp.mean((y - m) ** 2, axis=-1, keepdims=True)
        hh = (y - m) / jnp.sqrt(var + EPS) * g[...] + b[...]
        h_ref[...] = hh
        l = hh @ ws[...] + bs[...]
        lm = jnp.max(l, axis=-1, keepdims=True)
        el = jnp.exp(l - lm)
        z = jnp.sum(el, axis=-1, keepdims=True)
        prob = el / z
        pmax_ref[...] = 1.0 / z
        iota = lax.broadcasted_iota(jnp.int32, (RB, E), 1)
        ridx = jnp.min(jnp.where(l == lm, iota, E), axis=-1, keepdims=True)
        routes_ref[...] = ridx

        @pl.when(i == 0)
        def _():
            stats_ref[...] = jnp.zeros_like(stats_ref)

        onehot = (iota == ridx).astype(jnp.float32)
        stats_ref[0:1, :] += jnp.sum(onehot, axis=0, keepdims=True)
        stats_ref[1:2, :] += jnp.sum(prob, axis=0, keepdims=True)


def _attn(x, q, k, v, p):
    c = pl.BlockSpec((RB, 1), lambda i, h2: (i, 0))
    hp = lambda i, h2: (i, jnp.minimum(h2, H2 - 1))
    kp = lambda i, h2: (0, jnp.minimum(h2, H2 - 1))
    z = lambda i, h2: (0, 0)
    return pl.pallas_call(
        _attn_body,
        grid=(T // RB, H2 + 1),
        in_specs=[pl.BlockSpec((RB, D), lambda i, h2: (i, 0)),
                  pl.BlockSpec((RB, 2 * DH), hp),
                  pl.BlockSpec((T, 2 * DH), kp),
                  pl.BlockSpec((T, 2 * DH), kp),
                  pl.BlockSpec((D, D), z),
                  pl.BlockSpec((1, D), z),
                  pl.BlockSpec((1, D), z),
                  pl.BlockSpec((1, D), z),
                  pl.BlockSpec((D, E), z),
                  pl.BlockSpec((1, E), z)],
        out_specs=[pl.BlockSpec((RB, D), lambda i, h2: (i, 0)),
                   c, c, pl.BlockSpec((8, E), z)],
        out_shape=[jax.ShapeDtypeStruct((T, D), jnp.float32),
                   jax.ShapeDtypeStruct((T, 1), jnp.int32),
                   jax.ShapeDtypeStruct((T, 1), jnp.float32),
                   jax.ShapeDtypeStruct((8, E), jnp.float32)],
        scratch_shapes=[pltpu.VMEM((RB, D), jnp.float32)],
    )(x, q, k, v, p['Wo'], p['bo'].reshape(1, D), p['g_mha'].reshape(1, D),
      p['b_mha'].reshape(1, D), p['Ws'], p['bs'].reshape(1, E))


def _shift_sum(x, sbuf):
    for sh in (1, 2, 4, 8):
        sbuf[pl.ds(16, 16)] = x
        x = x + sbuf[pl.ds(16 - sh, 16)]
    return x


def _spread_max(x, sbuf):
    for sh in (1, 2, 4, 8):
        sbuf[pl.ds(16, 16)] = x
        x = jnp.maximum(x, sbuf[pl.ds(16 + sh, 16)])
    return x


def _onehot_mask(rv, e):
    return 1 - jnp.minimum(rv ^ e, 1)


def _route_build_body(routes_hbm, h_hbm, pmax_hbm,
                      d_hbm, blk_hbm, hs_hbm, ps_hbm,
                      routes_v, routes64_v, grid_v, acc_v, d_v, blk_v,
                      sbuf, rows_v, pm_v, shared, sem, sem2):
    c = lax.axis_index("c")
    s = lax.axis_index("s")
    w = s * 2 + c
    iota = lax.iota(jnp.int32, 16)
    zeros = jnp.zeros((16,), jnp.int32)
    sbuf[pl.ds(0, 16)] = zeros
    sbuf[pl.ds(32, 16)] = zeros

    cp_rows = pltpu.async_copy(h_hbm.at[pl.ds(w * 64, 64)], rows_v, sem)
    cp_pm = pltpu.async_copy(pmax_hbm.at[pl.ds(w * 64, 64)], pm_v, sem2)

    pltpu.sync_copy(routes_hbm.at[pl.ds(s * 128, 128)], routes_v)
    for half in range(2):
        for e in range(E):
            acc = zeros
            for vv in range(4):
                rv = routes_v[pl.ds(half * 64 + vv * 16, 16)]
                acc = acc + _onehot_mask(rv, e)
            acc_v[pl.ds(e * 16, 16)] = acc
        pltpu.sync_copy(acc_v, shared.at[pl.ds((s * 2 + half) * 128, 128)])
    plsc.subcore_barrier()
    pltpu.sync_copy(shared, grid_v)

    total = [jnp.zeros((16,), jnp.int32) for _ in range(E)]
    pre = [jnp.zeros((16,), jnp.int32) for _ in range(E)]
    for r in range(32):
        m_r = ((jnp.int32(r) - w) >> 31) & 1
        for e in range(E):
            acc = grid_v[pl.ds(r * 128 + e * 16, 16)]
            total[e] = total[e] + acc
            pre[e] = pre[e] + acc * m_r
    off_acc = jnp.zeros((16,), jnp.int32)
    offs = []
    incl = []
    for e in range(E):
        tot_e = _spread_max(_shift_sum(total[e], sbuf), sbuf)
        pre_e = _spread_max(_shift_sum(pre[e], sbuf), sbuf)
        pad_e = jnp.bitwise_and(tot_e + (BLK - 1), jnp.int32(-BLK))
        offs.append(off_acc + pre_e)
        off_acc = off_acc + pad_e
        incl.append(off_acc)

    @pl.when(w == 0)
    def _():
        for g2 in range(2):
            start = (iota + g2 * 16) * BLK
            eb = jnp.zeros((16,), jnp.int32)
            for e in range(E):
                eb = eb + 1 - (((start - incl[e]) >> 31) & 1)
            blk_v[pl.ds(g2 * 16, 16)] = jnp.minimum(eb, E - 1)
        pltpu.sync_copy(blk_v, blk_hbm)

    carry = [jnp.zeros((16,), jnp.int32) for _ in range(E)]
    pltpu.sync_copy(routes_hbm.at[pl.ds(w * 64, 64)], routes64_v)
    for vv in range(4):
        rv = routes64_v[pl.ds(vv * 16, 16)]
        d = jnp.zeros((16,), jnp.int32)
        for e in range(E):
            m = _onehot_mask(rv, e)
            cs = _shift_sum(m, sbuf)
            d = d + m * (offs[e] + carry[e] + cs - 1)
            carry[e] = carry[e] + _spread_max(cs, sbuf)
        d_v[pl.ds(vv * 16, 16)] = d
    pltpu.sync_copy(d_v, d_hbm.at[pl.ds(w * 64, 64)])

    cp_rows.wait()
    cp_pm.wait()
    pltpu.async_copy(rows_v, hs_hbm.at[d_v], sem).wait()
    pltpu.async_copy(pm_v, ps_hbm.at[d_v], sem2).wait()


def _route_build(routes, h, pmax):
    mesh = plsc.VectorSubcoreMesh(core_axis_name="c", subcore_axis_name="s")
    f = pl.kernel(
        _route_build_body,
        out_type=[jax.ShapeDtypeStruct((T,), jnp.int32),
                  jax.ShapeDtypeStruct((NBPAD,), jnp.int32),
                  jax.ShapeDtypeStruct((NP, D), jnp.float32),
                  jax.ShapeDtypeStruct((NP,), jnp.float32)],
        mesh=mesh,
        scratch_types=[
            pltpu.VMEM((128,), jnp.int32),
            pltpu.VMEM((64,), jnp.int32),
            pltpu.VMEM((4096,), jnp.int32),
            pltpu.VMEM((128,), jnp.int32),
            pltpu.VMEM((64,), jnp.int32),
            pltpu.VMEM((32,), jnp.int32),
            pltpu.VMEM((48,), jnp.int32),
            pltpu.VMEM((64, D), jnp.float32),
            pltpu.VMEM((64,), jnp.float32),
            pltpu.VMEM_SHARED((4096,), jnp.int32),
            pltpu.SemaphoreType.DMA,
            pltpu.SemaphoreType.DMA,
        ],
    )
    return f(routes, h, pmax)


def _ffn_body(eid_ref, hs_ref, ps_ref, w1_ref, b1_ref, w2_ref, b2_ref,
              ge_ref, be_ref, out_ref):
    x = hs_ref[...]
    a = jnp.maximum(x @ w1_ref[0] + b1_ref[0], 0.0)
    o = a @ w2_ref[0] + b2_ref[0]
    y = x + o
    m = jnp.mean(y, axis=-1, keepdims=True)
    var = jnp.mean((y - m) ** 2, axis=-1, keepdims=True)
    hn = (y - m) / jnp.sqrt(var + EPS) * ge_ref[0] + be_ref[0]
    out_ref[...] = hn * ps_ref[...]


def _ffn(blk_expert, hs, ps2, p):
    grid_spec = pltpu.PrefetchScalarGridSpec(
        num_scalar_prefetch=1,
        grid=(NB,),
        in_specs=[
            pl.BlockSpec((BLK, D), lambda b, eid: (b, 0)),
            pl.BlockSpec((BLK, 1), lambda b, eid: (b, 0)),
            pl.BlockSpec((1, D, DFF), lambda b, eid: (eid[b], 0, 0)),
            pl.BlockSpec((1, 1, DFF), lambda b, eid: (eid[b], 0, 0)),
            pl.BlockSpec((1, DFF, D), lambda b, eid: (eid[b], 0, 0)),
            pl.BlockSpec((1, 1, D), lambda b, eid: (eid[b], 0, 0)),
            pl.BlockSpec((1, 1, D), lambda b, eid: (eid[b], 0, 0)),
            pl.BlockSpec((1, 1, D), lambda b, eid: (eid[b], 0, 0)),
        ],
        out_specs=pl.BlockSpec((BLK, D), lambda b, eid: (b, 0)),
    )
    return pl.pallas_call(
        _ffn_body,
        grid_spec=grid_spec,
        out_shape=jax.ShapeDtypeStruct((NP, D), jnp.float32),
    )(blk_expert, hs, ps2, p['W1'], p['b1'].reshape(E, 1, DFF),
      p['W2'], p['b2'].reshape(E, 1, D),
      p['ge'].reshape(E, 1, D), p['be'].reshape(E, 1, D))


def _unperm_body(d_hbm, os_hbm, out_hbm, d_v, rows_v, sem):
    c = lax.axis_index("c")
    s = lax.axis_index("s")
    w = s * 2 + c
    pltpu.sync_copy(d_hbm.at[pl.ds(w * 64, 64)], d_v)
    pltpu.async_copy(os_hbm.at[d_v], rows_v, sem).wait()
    pltpu.sync_copy(rows_v, out_hbm.at[pl.ds(w * 64, 64)])


def _unperm(d, out_sorted):
    mesh = plsc.VectorSubcoreMesh(core_axis_name="c", subcore_axis_name="s")
    f = pl.kernel(
        _unperm_body,
        out_type=jax.ShapeDtypeStruct((T, D), jnp.float32),
        mesh=mesh,
        scratch_types=[
            pltpu.VMEM((64,), jnp.int32),
            pltpu.VMEM((64, D), jnp.float32),
            pltpu.SemaphoreType.DMA,
        ],
    )
    return f(d, out_sorted)


def kernel(x, params):
    p = params
    q, k, v = _proj(x, p)
    h, routes2d, pmax2d, stats = _attn(x, q, k, v, p)
    routes = routes2d.reshape(T)
    pmax = pmax2d.reshape(T)
    d, blk_expert, hs, ps = _route_build(routes, h, pmax)
    out_sorted = _ffn(blk_expert, hs, ps.reshape(NP, 1), p)
    out = _unperm(d, out_sorted)
    return out, stats[0], stats[1], jnp.int32(0), pmax
